# baseline (speedup 1.0000x reference)
import functools

import jax
import jax.numpy as jnp
from jax import lax
from jax.experimental import pallas as pl
from jax.experimental.pallas import tpu as pltpu
from jax.experimental.pallas import tpu_sc as plsc

BATCH = 4096
HIST = 200
DIM = 64
NUM_ROWS = BATCH * HIST
NW = 32
PER_W = NUM_ROWS // NW
CHUNK = 128
NCHUNK = PER_W // CHUNK


def _build_sc_gather():
    mesh = plsc.VectorSubcoreMesh(core_axis_name="c", subcore_axis_name="s")

    @functools.partial(
        pl.kernel,
        mesh=mesh,
        out_type=jax.ShapeDtypeStruct((NUM_ROWS, DIM), jnp.float32),
        scratch_types=[
            pltpu.VMEM((PER_W,), jnp.int32),
            pltpu.VMEM((CHUNK, DIM), jnp.float32),
            pltpu.SemaphoreType.DMA,
        ],
        compiler_params=pltpu.CompilerParams(use_tc_tiling_on_sc=False),
    )
    def emb_gather(table_hbm, idx_hbm, out_hbm, idx_v, rows_v, sem):
        wid = lax.axis_index("s") * 2 + lax.axis_index("c")
        base = wid * PER_W
        pltpu.sync_copy(idx_hbm.at[pl.ds(base, PER_W)], idx_v)

        def body(j, carry):
            off = j * CHUNK
            pltpu.async_copy(
                table_hbm.at[idx_v.at[pl.ds(off, CHUNK)]], rows_v, sem
            ).wait()
            pltpu.sync_copy(rows_v, out_hbm.at[pl.ds(base + off, CHUNK)])
            return carry

        lax.fori_loop(0, NCHUNK, body, 0)

    return emb_gather


_emb_gather = _build_sc_gather()


@jax.jit
def kernel(item_ids, table):
    idx_flat = item_ids.reshape(-1)
    out = _emb_gather(table, idx_flat)
    return out.reshape(item_ids.shape + (table.shape[1],))


# speedup vs baseline: 1.1144x; 1.1144x over previous
import functools

import jax
import jax.numpy as jnp
from jax import lax
from jax.experimental import pallas as pl
from jax.experimental.pallas import tpu as pltpu
from jax.experimental.pallas import tpu_sc as plsc

BATCH = 4096
HIST = 200
DIM = 64
NUM_ROWS = BATCH * HIST
NW = 32
PER_W = NUM_ROWS // NW
CHUNK = 128
NCHUNK = PER_W // CHUNK
NBUF = 8
NGROUP = NCHUNK // NBUF


def _build_sc_gather():
    mesh = plsc.VectorSubcoreMesh(core_axis_name="c", subcore_axis_name="s")

    @functools.partial(
        pl.kernel,
        mesh=mesh,
        out_type=jax.ShapeDtypeStruct((NUM_ROWS, DIM), jnp.float32),
        scratch_types=[
            pltpu.VMEM((PER_W,), jnp.int32),
            [pltpu.VMEM((CHUNK, DIM), jnp.float32) for _ in range(NBUF)],
            [pltpu.SemaphoreType.DMA for _ in range(NBUF)],
            [pltpu.SemaphoreType.DMA for _ in range(NBUF)],
        ],
        compiler_params=pltpu.CompilerParams(use_tc_tiling_on_sc=False),
    )
    def emb_gather(table_hbm, idx_hbm, out_hbm, idx_v, bufs, gsems, osems):
        wid = lax.axis_index("s") * 2 + lax.axis_index("c")
        base = wid * PER_W
        pltpu.sync_copy(idx_hbm.at[pl.ds(base, PER_W)], idx_v)

        def start_gather(chunk, b):
            pltpu.async_copy(
                table_hbm.at[idx_v.at[pl.ds(chunk * CHUNK, CHUNK)]],
                bufs[b],
                gsems[b],
            )

        def start_out(chunk, b):
            pltpu.async_copy(
                bufs[b], out_hbm.at[pl.ds(base + chunk * CHUNK, CHUNK)], osems[b]
            )

        def wait_gather(b):
            pltpu.make_async_copy(
                table_hbm.at[idx_v.at[pl.ds(0, CHUNK)]], bufs[b], gsems[b]
            ).wait()

        def wait_out(b):
            pltpu.make_async_copy(
                bufs[b], out_hbm.at[pl.ds(base, CHUNK)], osems[b]
            ).wait()

        for b in range(NBUF):
            start_gather(b, b)

        def body(g, carry):
            for b in range(NBUF):
                wait_gather(b)
                start_out(g * NBUF + b, b)

            @pl.when(g + 1 < NGROUP)
            def _():
                for b in range(NBUF):
                    wait_out(b)
                    start_gather((g + 1) * NBUF + b, b)

            return carry

        lax.fori_loop(0, NGROUP, body, 0)
        for b in range(NBUF):
            wait_out(b)

    return emb_gather


_emb_gather = _build_sc_gather()


@jax.jit
def kernel(item_ids, table):
    idx_flat = item_ids.reshape(-1)
    out = _emb_gather(table, idx_flat)
    return out.reshape(item_ids.shape + (table.shape[1],))
